# baseline (speedup 1.0000x reference)
import functools

import jax
import jax.numpy as jnp
from jax import lax
from jax.experimental import pallas as pl
from jax.experimental.pallas import tpu as pltpu
from jax.experimental.pallas import tpu_sc as plsc

HIDDEN = 1024
FFN = 4096
NE = 8
TOPK = 2
NTOK = 2048
NPAIR = NTOK * TOPK

BT = 256
MAXT = 24
P = MAXT * BT
FC = 1024
NF = FFN // FC

NC = 2
NS = 16
NW = NC * NS
GCH = 32


def _router_body(x_ref, wr_ref, i0_ref, i1_ref, w0_ref, w1_ref):
    x = x_ref[...]
    wr = wr_ref[...]
    logits = lax.dot_general(x, wr, (((1,), (1,)), ((), ())),
                             preferred_element_type=jnp.float32)
    col = lax.broadcasted_iota(jnp.int32, (NTOK, NE), 1)
    m0 = jnp.max(logits, axis=1, keepdims=True)
    i0 = jnp.min(jnp.where(logits == m0, col, NE), axis=1, keepdims=True)
    masked = jnp.where(col == i0, -jnp.inf, logits)
    m1 = jnp.max(masked, axis=1, keepdims=True)
    i1 = jnp.min(jnp.where(masked == m1, col, NE), axis=1, keepdims=True)
    e1 = jnp.exp(m1 - m0)
    denom = 1.0 + e1
    i0_ref[...] = i0
    i1_ref[...] = i1
    w0_ref[...] = 1.0 / denom
    w1_ref[...] = e1 / denom


def _router(xf, Wr):
    return pl.pallas_call(
        _router_body,
        out_shape=(
            jax.ShapeDtypeStruct((NTOK, 1), jnp.int32),
            jax.ShapeDtypeStruct((NTOK, 1), jnp.int32),
            jax.ShapeDtypeStruct((NTOK, 1), jnp.float32),
            jax.ShapeDtypeStruct((NTOK, 1), jnp.float32),
        ),
    )(xf, Wr)


def _dispatch(i0, i1, w0, w1):
    flat_e = jnp.concatenate([i0, i1], axis=1).reshape(-1)
    flat_w = jnp.concatenate([w0, w1], axis=1).reshape(-1)
    onehot = (flat_e[:, None] == jnp.arange(NE)[None, :]).astype(jnp.int32)
    csum = jnp.cumsum(onehot, axis=0)
    rank = jnp.take_along_axis(csum, flat_e[:, None], axis=1)[:, 0] - 1
    counts = csum[-1]
    padded = ((counts + BT - 1) // BT) * BT
    cum_padded = jnp.cumsum(padded)
    pad_off = cum_padded - padded
    slot = pad_off[flat_e] + rank
    token_for_slot = jnp.zeros((P,), jnp.int32).at[slot].set(
        jnp.arange(NPAIR, dtype=jnp.int32) // TOPK)
    w_slot = jnp.zeros((P,), jnp.float32).at[slot].set(flat_w)
    s0 = slot[0::TOPK].astype(jnp.int32)
    s1 = slot[1::TOPK].astype(jnp.int32)
    num_active = (cum_padded[-1] // BT).astype(jnp.int32)
    tile_base = jnp.arange(MAXT, dtype=jnp.int32) * BT
    raw = jnp.searchsorted(cum_padded, tile_base, side='right').astype(jnp.int32)
    last_e = jnp.take(raw, num_active - 1)
    active = tile_base < cum_padded[-1]
    tile_expert = jnp.where(active, jnp.minimum(raw, NE - 1), last_e)
    tile_valid = active.astype(jnp.int32)
    return token_for_slot, w_slot, s0, s1, tile_expert, tile_valid


def _sc_gather_body(x_hbm, tok_hbm, xs_hbm, idx_v, rows_v, sem):
    wid = lax.axis_index("s") * NC + lax.axis_index("c")
    per_w = P // NW
    for c in range(per_w // GCH):
        base = wid * per_w + c * GCH
        pltpu.sync_copy(tok_hbm.at[pl.ds(base, GCH)], idx_v)
        pltpu.async_copy(x_hbm.at[idx_v], rows_v, sem).wait()
        pltpu.sync_copy(rows_v, xs_hbm.at[pl.ds(base, GCH)])


def _sc_gather(xb32, token_for_slot):
    mesh = plsc.VectorSubcoreMesh(core_axis_name="c", subcore_axis_name="s")
    return pl.kernel(
        _sc_gather_body,
        mesh=mesh,
        out_type=jax.ShapeDtypeStruct((P, HIDDEN // 2), jnp.int32),
        scratch_types=[
            pltpu.VMEM((GCH,), jnp.int32),
            pltpu.VMEM((GCH, HIDDEN // 2), jnp.int32),
            pltpu.SemaphoreType.DMA,
        ],
    )(xb32, token_for_slot)


def _ffn_body(te_ref, va_ref, xs_ref, w1_ref, b1_ref, w2_ref, b2_ref, ws_ref,
              out_ref):
    j = pl.program_id(0)

    @pl.when(va_ref[j] == 1)
    def _():
        xs = xs_ref[...]
        h = lax.dot_general(xs, w1_ref[0], (((1,), (1,)), ((), ())),
                            preferred_element_type=jnp.float32)
        h = h + b1_ref[0]
        h = 0.5 * h * (1.0 + lax.erf(h * 0.7071067811865476))
        y = lax.dot_general(h, w2_ref[0], (((1,), (1,)), ((), ())),
                            preferred_element_type=jnp.float32)
        out_ref[...] = (y + b2_ref[0]) * ws_ref[...]


def _ffn(xs, W1, b1, W2, b2, w_slot, tile_expert, tile_valid):
    grid_spec = pltpu.PrefetchScalarGridSpec(
        num_scalar_prefetch=2,
        grid=(MAXT,),
        in_specs=[
            pl.BlockSpec((BT, HIDDEN), lambda j, te, va: (j, 0)),
            pl.BlockSpec((1, FFN, HIDDEN), lambda j, te, va: (te[j], 0, 0)),
            pl.BlockSpec((1, 1, FFN), lambda j, te, va: (te[j], 0, 0)),
            pl.BlockSpec((1, HIDDEN, FFN), lambda j, te, va: (te[j], 0, 0)),
            pl.BlockSpec((1, 1, HIDDEN), lambda j, te, va: (te[j], 0, 0)),
            pl.BlockSpec((BT, 1), lambda j, te, va: (j, 0)),
        ],
        out_specs=pl.BlockSpec((BT, HIDDEN), lambda j, te, va: (j, 0)),
    )
    return pl.pallas_call(
        _ffn_body,
        grid_spec=grid_spec,
        out_shape=jax.ShapeDtypeStruct((P, HIDDEN), jnp.float32),
        compiler_params=pltpu.CompilerParams(
            vmem_limit_bytes=110 * 1024 * 1024,
            dimension_semantics=("arbitrary",),
        ),
    )(tile_expert, tile_valid, xs, W1.astype(jnp.bfloat16),
      b1.reshape(NE, 1, FFN), W2.astype(jnp.bfloat16),
      b2.reshape(NE, 1, HIDDEN), w_slot.reshape(P, 1))


CCH = 32


def _sc_combine_body(ys_hbm, s0_hbm, s1_hbm, out_hbm, i0_v, i1_v, r0, r1, sem):
    wid = lax.axis_index("s") * NC + lax.axis_index("c")
    per_w = NTOK // NW
    for c in range(per_w // CCH):
        base = wid * per_w + c * CCH
        pltpu.sync_copy(s0_hbm.at[pl.ds(base, CCH)], i0_v)
        h0 = pltpu.async_copy(ys_hbm.at[i0_v], r0, sem)
        pltpu.sync_copy(s1_hbm.at[pl.ds(base, CCH)], i1_v)
        h1 = pltpu.async_copy(ys_hbm.at[i1_v], r1, sem)
        h0.wait()
        h1.wait()

        def body(i, carry):
            t = i // (HIDDEN // 16)
            off = (i % (HIDDEN // 16)) * 16
            a = r0[t, pl.ds(off, 16)]
            b = r1[t, pl.ds(off, 16)]
            r0[t, pl.ds(off, 16)] = a + b
            return carry

        lax.fori_loop(0, CCH * (HIDDEN // 16), body, 0)
        pltpu.sync_copy(r0, out_hbm.at[pl.ds(base, CCH)])


def _sc_combine(ys, s0, s1):
    mesh = plsc.VectorSubcoreMesh(core_axis_name="c", subcore_axis_name="s")
    return pl.kernel(
        _sc_combine_body,
        mesh=mesh,
        out_type=jax.ShapeDtypeStruct((NTOK, HIDDEN), jnp.float32),
        scratch_types=[
            pltpu.VMEM((CCH,), jnp.int32),
            pltpu.VMEM((CCH,), jnp.int32),
            pltpu.VMEM((CCH, HIDDEN), jnp.float32),
            pltpu.VMEM((CCH, HIDDEN), jnp.float32),
            pltpu.SemaphoreType.DMA,
        ],
    )(ys, s0, s1)


def kernel(x, Wr, W1, b1, W2, b2):
    Bn, Sn, Dn = x.shape
    xf = x.reshape(Bn * Sn, Dn)
    i0, i1, w0, w1 = _router(xf, Wr)
    token_for_slot, w_slot, s0, s1, tile_expert, tile_valid = _dispatch(
        i0, i1, w0, w1)
    xb32 = lax.bitcast_convert_type(
        xf.astype(jnp.bfloat16).reshape(Bn * Sn, Dn // 2, 2), jnp.int32)
    xs32 = _sc_gather(xb32, token_for_slot)
    xs = lax.bitcast_convert_type(xs32, jnp.bfloat16).reshape(P, HIDDEN)
    ys = _ffn(xs, W1, b1, W2, b2, w_slot, tile_expert, tile_valid)
    out = _sc_combine(ys, s0, s1)
    return out.reshape(Bn, Sn, Dn)


# speedup vs baseline: 1.8444x; 1.8444x over previous
import functools

import jax
import jax.numpy as jnp
from jax import lax
from jax.experimental import pallas as pl
from jax.experimental.pallas import tpu as pltpu
from jax.experimental.pallas import tpu_sc as plsc

HIDDEN = 1024
FFN = 4096
NE = 8
TOPK = 2
NTOK = 2048
NPAIR = NTOK * TOPK

BT = 256
MAXT = 24
P = MAXT * BT
FC = 1024
NF = FFN // FC

NC = 2
NS = 16
NW = NC * NS
GCH = 32


def _router_body(x_ref, wr_ref, i0_ref, i1_ref, w0_ref, w1_ref):
    x = x_ref[...]
    wr = wr_ref[...]
    logits = lax.dot_general(x, wr, (((1,), (1,)), ((), ())),
                             preferred_element_type=jnp.float32)
    col = lax.broadcasted_iota(jnp.int32, (NTOK, NE), 1)
    m0 = jnp.max(logits, axis=1, keepdims=True)
    i0 = jnp.min(jnp.where(logits == m0, col, NE), axis=1, keepdims=True)
    masked = jnp.where(col == i0, -jnp.inf, logits)
    m1 = jnp.max(masked, axis=1, keepdims=True)
    i1 = jnp.min(jnp.where(masked == m1, col, NE), axis=1, keepdims=True)
    e1 = jnp.exp(m1 - m0)
    denom = 1.0 + e1
    i0_ref[...] = i0
    i1_ref[...] = i1
    w0_ref[...] = 1.0 / denom
    w1_ref[...] = e1 / denom


def _router(xf, Wr):
    return pl.pallas_call(
        _router_body,
        out_shape=(
            jax.ShapeDtypeStruct((NTOK, 1), jnp.int32),
            jax.ShapeDtypeStruct((NTOK, 1), jnp.int32),
            jax.ShapeDtypeStruct((NTOK, 1), jnp.float32),
            jax.ShapeDtypeStruct((NTOK, 1), jnp.float32),
        ),
    )(xf, Wr)


def _dispatch(i0, i1, w0, w1):
    flat_e = jnp.concatenate([i0, i1], axis=1).reshape(-1)
    flat_w = jnp.concatenate([w0, w1], axis=1).reshape(-1)
    onehot = (flat_e[:, None] == jnp.arange(NE)[None, :]).astype(jnp.int32)
    csum = jnp.cumsum(onehot, axis=0)
    rank = jnp.take_along_axis(csum, flat_e[:, None], axis=1)[:, 0] - 1
    counts = csum[-1]
    padded = ((counts + BT - 1) // BT) * BT
    cum_padded = jnp.cumsum(padded)
    pad_off = cum_padded - padded
    slot = pad_off[flat_e] + rank
    w_slot = jnp.zeros((P,), jnp.float32).at[slot].set(flat_w)
    s0 = slot[0::TOPK].astype(jnp.int32)
    s1 = slot[1::TOPK].astype(jnp.int32)
    num_active = (cum_padded[-1] // BT).astype(jnp.int32)
    tile_base = jnp.arange(MAXT, dtype=jnp.int32) * BT
    raw = jnp.searchsorted(cum_padded, tile_base, side='right').astype(jnp.int32)
    last_e = jnp.take(raw, num_active - 1)
    active = tile_base < cum_padded[-1]
    tile_expert = jnp.where(active, jnp.minimum(raw, NE - 1), last_e)
    tile_valid = active.astype(jnp.int32)
    return w_slot, s0, s1, tile_expert, tile_valid


TPW = NTOK // NW


def _sc_scatter_body(x_hbm, s0_hbm, s1_hbm, xs_hbm, i0_v, i1_v, rows_v,
                     s0sem, s1sem):
    wid = lax.axis_index("s") * NC + lax.axis_index("c")
    pltpu.sync_copy(x_hbm.at[pl.ds(wid * TPW, TPW)], rows_v)
    pltpu.sync_copy(s0_hbm.at[wid], i0_v)
    pltpu.sync_copy(s1_hbm.at[wid], i1_v)
    h0 = pltpu.async_copy(rows_v, xs_hbm.at[i0_v], s0sem)
    h1 = pltpu.async_copy(rows_v, xs_hbm.at[i1_v], s1sem)
    h0.wait()
    h1.wait()


def _sc_scatter(xf, s0, s1):
    mesh = plsc.VectorSubcoreMesh(core_axis_name="c", subcore_axis_name="s")
    return pl.kernel(
        _sc_scatter_body,
        mesh=mesh,
        out_type=jax.ShapeDtypeStruct((P, HIDDEN), jnp.float32),
        scratch_types=[
            pltpu.VMEM((TPW,), jnp.int32),
            pltpu.VMEM((TPW,), jnp.int32),
            pltpu.VMEM((TPW, HIDDEN), jnp.float32),
            pltpu.SemaphoreType.DMA,
            pltpu.SemaphoreType.DMA,
        ],
    )(xf, s0.reshape(NW, TPW), s1.reshape(NW, TPW))


def _ffn_body(te_ref, va_ref, xs_ref, w1_ref, b1_ref, w2_ref, b2_ref, ws_ref,
              out_ref):
    j = pl.program_id(0)

    @pl.when(va_ref[j] == 1)
    def _():
        xs = xs_ref[...]
        h = lax.dot_general(xs, w1_ref[0], (((1,), (1,)), ((), ())),
                            preferred_element_type=jnp.float32)
        h = h + b1_ref[0]
        h = 0.5 * h * (1.0 + lax.erf(h * 0.7071067811865476))
        y = lax.dot_general(h, w2_ref[0], (((1,), (1,)), ((), ())),
                            preferred_element_type=jnp.float32)
        out_ref[...] = (y + b2_ref[0]) * ws_ref[...]


def _ffn(xs, W1, b1, W2, b2, w_slot, tile_expert, tile_valid):
    grid_spec = pltpu.PrefetchScalarGridSpec(
        num_scalar_prefetch=2,
        grid=(MAXT,),
        in_specs=[
            pl.BlockSpec((BT, HIDDEN), lambda j, te, va: (j, 0)),
            pl.BlockSpec((1, FFN, HIDDEN), lambda j, te, va: (te[j], 0, 0)),
            pl.BlockSpec((1, 1, FFN), lambda j, te, va: (te[j], 0, 0)),
            pl.BlockSpec((1, HIDDEN, FFN), lambda j, te, va: (te[j], 0, 0)),
            pl.BlockSpec((1, 1, HIDDEN), lambda j, te, va: (te[j], 0, 0)),
            pl.BlockSpec((BT, 1), lambda j, te, va: (j, 0)),
        ],
        out_specs=pl.BlockSpec((BT, HIDDEN), lambda j, te, va: (j, 0)),
    )
    return pl.pallas_call(
        _ffn_body,
        grid_spec=grid_spec,
        out_shape=jax.ShapeDtypeStruct((P, HIDDEN), jnp.float32),
        compiler_params=pltpu.CompilerParams(
            vmem_limit_bytes=110 * 1024 * 1024,
            dimension_semantics=("arbitrary",),
        ),
    )(tile_expert, tile_valid, xs, W1.astype(jnp.bfloat16),
      b1.reshape(NE, 1, FFN), W2.astype(jnp.bfloat16),
      b2.reshape(NE, 1, HIDDEN), w_slot.reshape(P, 1))


CCH = 32


def _sc_combine_body(ys_hbm, s0_hbm, s1_hbm, out_hbm, i0_v, i1_v, r0, r1, sem):
    wid = lax.axis_index("s") * NC + lax.axis_index("c")
    per_w = NTOK // NW
    for c in range(per_w // CCH):
        base = wid * per_w + c * CCH
        pltpu.sync_copy(s0_hbm.at[pl.ds(base, CCH)], i0_v)
        h0 = pltpu.async_copy(ys_hbm.at[i0_v], r0, sem)
        pltpu.sync_copy(s1_hbm.at[pl.ds(base, CCH)], i1_v)
        h1 = pltpu.async_copy(ys_hbm.at[i1_v], r1, sem)
        h0.wait()
        h1.wait()

        def body(i, carry):
            t = i // (HIDDEN // 16)
            off = (i % (HIDDEN // 16)) * 16
            a = r0[t, pl.ds(off, 16)]
            b = r1[t, pl.ds(off, 16)]
            r0[t, pl.ds(off, 16)] = a + b
            return carry

        lax.fori_loop(0, CCH * (HIDDEN // 16), body, 0)
        pltpu.sync_copy(r0, out_hbm.at[pl.ds(base, CCH)])


def _sc_combine(ys, s0, s1):
    mesh = plsc.VectorSubcoreMesh(core_axis_name="c", subcore_axis_name="s")
    return pl.kernel(
        _sc_combine_body,
        mesh=mesh,
        out_type=jax.ShapeDtypeStruct((NTOK, HIDDEN), jnp.float32),
        scratch_types=[
            pltpu.VMEM((CCH,), jnp.int32),
            pltpu.VMEM((CCH,), jnp.int32),
            pltpu.VMEM((CCH, HIDDEN), jnp.float32),
            pltpu.VMEM((CCH, HIDDEN), jnp.float32),
            pltpu.SemaphoreType.DMA,
        ],
    )(ys, s0, s1)


def kernel(x, Wr, W1, b1, W2, b2):
    Bn, Sn, Dn = x.shape
    xf = x.reshape(Bn * Sn, Dn)
    i0, i1, w0, w1 = _router(xf, Wr)
    w_slot, s0, s1, tile_expert, tile_valid = _dispatch(i0, i1, w0, w1)
    xs = _sc_scatter(xf, s0, s1)
    ys = _ffn(xs, W1, b1, W2, b2, w_slot, tile_expert, tile_valid)
    out = _sc_combine(ys, s0, s1)
    return out.reshape(Bn, Sn, Dn)
